# baseline (speedup 1.0000x reference)
import functools

import jax
import jax.numpy as jnp
from jax import lax
from jax.experimental import pallas as pl
from jax.experimental.pallas import tpu as pltpu
from jax.experimental.pallas import tpu_sc as plsc

_NC = 2
_NS = 16
_NW = _NC * _NS
_C = 64
_LANES = 16
_NBUF = 4
_MBUF = 8


@functools.partial(jax.jit, static_argnames=("n", "d", "cpw"))
def _sc_aggregate(src, dst, ew, x2d, *, n, d, cpw):
    rpt = (n // _NS) // 8 * 8
    tail = n - _NS * rpt
    dreg = d // _LANES

    mesh = plsc.VectorSubcoreMesh(core_axis_name="c", subcore_axis_name="s")

    @functools.partial(
        pl.kernel,
        out_type=jax.ShapeDtypeStruct((_NC, n, d), jnp.float32),
        mesh=mesh,
        scratch_types=[
            pltpu.VMEM_SHARED((n, d), jnp.float32),
            pltpu.VMEM((_MBUF, _C), jnp.int32),
            pltpu.VMEM((_MBUF, _C), jnp.int32),
            pltpu.VMEM((_MBUF, _C), jnp.float32),
            pltpu.VMEM((_NBUF, _C, d), jnp.float32),
            [pltpu.SemaphoreType.DMA] * _MBUF,
            [pltpu.SemaphoreType.DMA] * _NBUF,
            [pltpu.SemaphoreType.DMA] * _NBUF,
        ],
    )
    def k(src_hbm, dst_hbm, ew_hbm, x_hbm, out_hbm, accum, sidx, didx, wring, rows, msem, gsem, ssem):
        cid = lax.axis_index("c")
        sid = lax.axis_index("s")
        wid = sid * _NC + cid

        def fetch_meta(c, m):
            off = (c * _NW + wid) * _C
            pltpu.async_copy(src_hbm.at[pl.ds(off, _C)], sidx.at[m], msem[m])
            pltpu.async_copy(dst_hbm.at[pl.ds(off, _C)], didx.at[m], msem[m])
            pltpu.async_copy(ew_hbm.at[pl.ds(off, _C)], wring.at[m], msem[m])

        def wait_meta(m):
            for ref in (sidx, didx, wring):
                pltpu.make_async_copy(
                    src_hbm.at[pl.ds(0, _C)], ref.at[m], msem[m]
                ).wait()

        zero16 = jnp.zeros((_LANES,), jnp.float32)
        zbuf = rows.at[0]

        def zbody(i, _):
            for r in range(dreg):
                zbuf[i, pl.ds(r * _LANES, _LANES)] = zero16
            return 0

        lax.fori_loop(0, _C, zbody, 0)

        base_row = sid * rpt
        nfull = rpt // _C
        rem = rpt - nfull * _C
        for j in range(nfull):
            pltpu.sync_copy(zbuf, accum.at[pl.ds(base_row + j * _C, _C)])
        if rem:
            pltpu.sync_copy(
                zbuf.at[pl.ds(0, rem)],
                accum.at[pl.ds(base_row + nfull * _C, rem)],
            )
        if tail:

            @pl.when(sid == _NS - 1)
            def _():
                pltpu.sync_copy(
                    zbuf.at[pl.ds(0, tail)], accum.at[pl.ds(_NS * rpt, tail)]
                )

        for c in range(min(_MBUF - 2, cpw)):
            fetch_meta(c, c)
        for c in range(min(2, cpw)):
            wait_meta(c)
            pltpu.async_copy(x_hbm.at[sidx.at[c]], rows.at[c], gsem[c])

        plsc.subcore_barrier()

        def chunk(c, b, m):
            b2 = (b + 2) % _NBUF
            m2 = (m + 2) % _MBUF
            m6 = (m + 6) % _MBUF

            @pl.when(c >= 2)
            def _():
                pltpu.make_async_copy(
                    rows.at[b2], accum.at[didx.at[m]], ssem[b2]
                ).wait()

            @pl.when(c + 6 < cpw)
            def _():
                fetch_meta(c + 6, m6)

            @pl.when(c + 2 < cpw)
            def _():
                wait_meta(m2)
                pltpu.async_copy(x_hbm.at[sidx.at[m2]], rows.at[b2], gsem[b2])

            pltpu.make_async_copy(
                x_hbm.at[sidx.at[m]], rows.at[b], gsem[b]
            ).wait()

            def scale(q, _):
                wvec = wring[m, pl.ds(q * _LANES, _LANES)]
                for l in range(_LANES):
                    w = jnp.full((_LANES,), wvec[l], jnp.float32)
                    e = q * _LANES + l
                    for r in range(dreg):
                        sl = pl.ds(r * _LANES, _LANES)
                        rows[b, e, sl] = rows[b, e, sl] * w
                return 0

            lax.fori_loop(0, _C // _LANES, scale, 0)
            pltpu.async_copy(rows.at[b], accum.at[didx.at[m]], ssem[b], add=True)

        per = _MBUF

        def biggroup(t, _):
            c0 = t * per
            for j in range(per):
                chunk(c0 + j, j % _NBUF, j % _MBUF)
            return 0

        with jax.named_scope("edge_loop"):
            lax.fori_loop(0, cpw // per, biggroup, 0)

        for c in (cpw - 2, cpw - 1):
            pltpu.make_async_copy(
                rows.at[c % _NBUF], accum.at[didx.at[c % _MBUF]], ssem[c % _NBUF]
            ).wait()

        plsc.subcore_barrier()

        pltpu.sync_copy(
            accum.at[pl.ds(base_row, rpt)],
            out_hbm.at[cid, pl.ds(base_row, rpt)],
        )
        if tail:

            @pl.when(sid == _NS - 1)
            def _():
                pltpu.sync_copy(
                    accum.at[pl.ds(_NS * rpt, tail)],
                    out_hbm.at[cid, pl.ds(_NS * rpt, tail)],
                )

    return k(src, dst, ew, x2d)


def _tc_combine(partials, w, *, n, d_in, d_out, blk):
    def body(p_ref, w_ref, o_ref):
        p = p_ref[0] + p_ref[1]
        o_ref[...] = jnp.maximum(
            jnp.dot(p, w_ref[...], preferred_element_type=jnp.float32), 0.0
        )

    return pl.pallas_call(
        body,
        grid=(n // blk,),
        in_specs=[
            pl.BlockSpec((2, blk, d_in), lambda i: (0, i, 0)),
            pl.BlockSpec((d_in, d_out), lambda i: (0, 0)),
        ],
        out_specs=pl.BlockSpec((blk, d_out), lambda i: (i, 0)),
        out_shape=jax.ShapeDtypeStruct((n, d_out), jnp.float32),
    )(partials, w)


def kernel(x, edge_index, edge_weight, W):
    b, n, d_in = x.shape
    e = edge_index.shape[1]
    d_out = W.shape[1]

    cpw = -(-e // (_NW * _C))
    cpw = -(-cpw // _MBUF) * _MBUF
    e_pad = _NW * cpw * _C
    pad = e_pad - e
    src = edge_index[0]
    dst = edge_index[1]
    ew = edge_weight
    if pad:
        zi = (jnp.arange(pad, dtype=jnp.int32)) % n
        src = jnp.concatenate([src, zi])
        dst = jnp.concatenate([dst, zi])
        ew = jnp.concatenate([ew, jnp.zeros((pad,), jnp.float32)])

    blk = 1000 if n % 1000 == 0 else n
    outs = []
    for i in range(b):
        partials = _sc_aggregate(src, dst, ew, x[i], n=n, d=d_in, cpw=cpw)
        outs.append(_tc_combine(partials, W, n=n, d_in=d_in, d_out=d_out, blk=blk))
    if b == 1:
        return outs[0].reshape(1, n, d_out)
    return jnp.stack(outs, axis=0)


# speedup vs baseline: 1.0989x; 1.0989x over previous
import functools

import jax
import jax.numpy as jnp
from jax import lax
from jax.experimental import pallas as pl
from jax.experimental.pallas import tpu as pltpu
from jax.experimental.pallas import tpu_sc as plsc

_NC = 2
_NS = 16
_NW = _NC * _NS
_C = 64
_LANES = 16
_NBUF = 4
_MBUF = 8


@functools.partial(jax.jit, static_argnames=("n", "d", "cpw"))
def _sc_aggregate(src, dst, ew, x2d, *, n, d, cpw):
    rpt = (n // _NS) // 8 * 8
    tail = n - _NS * rpt
    dreg = d // _LANES

    mesh = plsc.VectorSubcoreMesh(core_axis_name="c", subcore_axis_name="s")

    @functools.partial(
        pl.kernel,
        out_type=jax.ShapeDtypeStruct((_NC, n, d), jnp.float32),
        mesh=mesh,
        scratch_types=[
            pltpu.VMEM_SHARED((n, d), jnp.float32),
            pltpu.VMEM((_MBUF, _C), jnp.int32),
            pltpu.VMEM((_MBUF, _C), jnp.int32),
            pltpu.VMEM((_MBUF, _C), jnp.float32),
            pltpu.VMEM((_NBUF, _C, d), jnp.float32),
            [pltpu.SemaphoreType.DMA] * _MBUF,
            [pltpu.SemaphoreType.DMA] * _NBUF,
            [pltpu.SemaphoreType.DMA] * _NBUF,
        ],
    )
    def k(src_hbm, dst_hbm, ew_hbm, x_hbm, out_hbm, accum, sidx, didx, wring, rows, msem, gsem, ssem):
        cid = lax.axis_index("c")
        sid = lax.axis_index("s")
        wid = sid * _NC + cid

        def fetch_meta(c, m):
            off = (c * _NW + wid) * _C
            pltpu.async_copy(src_hbm.at[pl.ds(off, _C)], sidx.at[m], msem[m])
            pltpu.async_copy(dst_hbm.at[pl.ds(off, _C)], didx.at[m], msem[m])
            pltpu.async_copy(ew_hbm.at[pl.ds(off, _C)], wring.at[m], msem[m])

        def wait_meta(m):
            for ref in (sidx, didx, wring):
                pltpu.make_async_copy(
                    src_hbm.at[pl.ds(0, _C)], ref.at[m], msem[m]
                ).wait()

        zero16 = jnp.zeros((_LANES,), jnp.float32)
        zbuf = rows.at[0]

        def zbody(i, _):
            for r in range(dreg):
                zbuf[i, pl.ds(r * _LANES, _LANES)] = zero16
            return 0

        lax.fori_loop(0, _C, zbody, 0)

        base_row = sid * rpt
        nfull = rpt // _C
        rem = rpt - nfull * _C
        for j in range(nfull):
            pltpu.sync_copy(zbuf, accum.at[pl.ds(base_row + j * _C, _C)])
        if rem:
            pltpu.sync_copy(
                zbuf.at[pl.ds(0, rem)],
                accum.at[pl.ds(base_row + nfull * _C, rem)],
            )
        if tail:

            @pl.when(sid == _NS - 1)
            def _():
                pltpu.sync_copy(
                    zbuf.at[pl.ds(0, tail)], accum.at[pl.ds(_NS * rpt, tail)]
                )

        for c in range(min(_MBUF - 2, cpw)):
            fetch_meta(c, c)
        for c in range(min(2, cpw)):
            wait_meta(c)
            pltpu.async_copy(x_hbm.at[sidx.at[c]], rows.at[c], gsem[c])

        plsc.subcore_barrier()

        def chunk(c, b, m):
            b2 = (b + 2) % _NBUF
            m2 = (m + 2) % _MBUF
            m6 = (m + 6) % _MBUF

            @pl.when(c >= 2)
            def _():
                pltpu.make_async_copy(
                    rows.at[b2], accum.at[didx.at[m]], ssem[b2]
                ).wait()

            @pl.when(c + 6 < cpw)
            def _():
                fetch_meta(c + 6, m6)

            @pl.when(c + 2 < cpw)
            def _():
                wait_meta(m2)
                pltpu.async_copy(x_hbm.at[sidx.at[m2]], rows.at[b2], gsem[b2])

            pltpu.make_async_copy(
                x_hbm.at[sidx.at[m]], rows.at[b], gsem[b]
            ).wait()

            def scale(q, _):
                wvec = wring[m, pl.ds(q * _LANES, _LANES)]
                for l in range(_LANES):
                    w = jnp.full((_LANES,), wvec[l], jnp.float32)
                    e = q * _LANES + l
                    for r in range(dreg):
                        sl = pl.ds(r * _LANES, _LANES)
                        rows[b, e, sl] = rows[b, e, sl] * w
                return 0

            lax.fori_loop(0, _C // _LANES, scale, 0)
            pltpu.async_copy(rows.at[b], accum.at[didx.at[m]], ssem[b], add=True)

        per = _MBUF

        def biggroup(t, _):
            c0 = t * per
            for j in range(per):
                chunk(c0 + j, j % _NBUF, j % _MBUF)
            return 0

        with jax.named_scope("edge_loop"):
            lax.fori_loop(0, cpw // per, biggroup, 0)

        for c in (cpw - 2, cpw - 1):
            pltpu.make_async_copy(
                rows.at[c % _NBUF], accum.at[didx.at[c % _MBUF]], ssem[c % _NBUF]
            ).wait()

        plsc.subcore_barrier()

        pltpu.sync_copy(
            accum.at[pl.ds(base_row, rpt)],
            out_hbm.at[cid, pl.ds(base_row, rpt)],
        )
        if tail:

            @pl.when(sid == _NS - 1)
            def _():
                pltpu.sync_copy(
                    accum.at[pl.ds(_NS * rpt, tail)],
                    out_hbm.at[cid, pl.ds(_NS * rpt, tail)],
                )

    return k(src, dst, ew, x2d)


def _tc_prep(edge_index, ew, *, e, e_pad, n):
    pad = e_pad - e

    def body(ei_ref, w_ref, s_ref, d_ref, o_ref):
        s_ref[pl.ds(0, e)] = ei_ref[0, :]
        d_ref[pl.ds(0, e)] = ei_ref[1, :]
        o_ref[pl.ds(0, e)] = w_ref[...]
        if pad:
            padrow = jnp.remainder(
                lax.broadcasted_iota(jnp.int32, (pad,), 0), n
            )
            s_ref[pl.ds(e, pad)] = padrow
            d_ref[pl.ds(e, pad)] = padrow
            o_ref[pl.ds(e, pad)] = jnp.zeros((pad,), jnp.float32)

    out = jax.ShapeDtypeStruct((e_pad,), jnp.int32)
    return pl.pallas_call(
        body,
        out_shape=[out, out, jax.ShapeDtypeStruct((e_pad,), jnp.float32)],
    )(edge_index, ew)


def _tc_combine(partials, w, *, n, d_in, d_out, blk):
    def body(p_ref, w_ref, o_ref):
        p = p_ref[0] + p_ref[1]
        o_ref[...] = jnp.maximum(
            jnp.dot(p, w_ref[...], preferred_element_type=jnp.float32), 0.0
        )

    return pl.pallas_call(
        body,
        grid=(n // blk,),
        in_specs=[
            pl.BlockSpec((2, blk, d_in), lambda i: (0, i, 0)),
            pl.BlockSpec((d_in, d_out), lambda i: (0, 0)),
        ],
        out_specs=pl.BlockSpec((blk, d_out), lambda i: (i, 0)),
        out_shape=jax.ShapeDtypeStruct((n, d_out), jnp.float32),
    )(partials, w)


def kernel(x, edge_index, edge_weight, W):
    b, n, d_in = x.shape
    e = edge_index.shape[1]
    d_out = W.shape[1]

    cpw = -(-e // (_NW * _C))
    cpw = -(-cpw // _MBUF) * _MBUF
    e_pad = _NW * cpw * _C

    src, dst, ew = _tc_prep(edge_index, edge_weight, e=e, e_pad=e_pad, n=n)

    blk = 2000 if n % 2000 == 0 else n
    outs = []
    for i in range(b):
        partials = _sc_aggregate(src, dst, ew, x[i], n=n, d=d_in, cpw=cpw)
        outs.append(_tc_combine(partials, W, n=n, d_in=d_in, d_out=d_out, blk=blk))
    if b == 1:
        return outs[0].reshape(1, n, d_out)
    return jnp.stack(outs, axis=0)
